# Initial kernel scaffold; baseline (speedup 1.0000x reference)
import jax
import jax.numpy as jnp
from jax.experimental import pallas as pl


def kernel(x, edge_index, edge_attr, W1, b1, W2, b2):
    raise NotImplementedError("write your pallas kernel here")


# speedup vs baseline: 2.1482x; 2.1482x over previous
import functools

import jax
import jax.numpy as jnp
from jax import lax
from jax.experimental import pallas as pl
from jax.experimental.pallas import tpu as pltpu
from jax.experimental.pallas import tpu_sc as plsc

D = 128
D_EDGE = 4
NC = 2
NS = 16
L = 16
NW = NC * NS
CHUNK = 128
AGG_PAD_ROWS = 10112


def _proj_body(x_ref, w_ref, b1_ref, ps_ref, pd_ref):
    xv = x_ref[...]
    p = jnp.dot(xv, w_ref[...], preferred_element_type=jnp.float32)
    ps_ref[...] = p[:, :D]
    pd_ref[...] = p[:, D:] + b1_ref[...]


def _eproj_body(ea_ref, w_ref, o_ref):
    o_ref[...] = jnp.dot(ea_ref[...], w_ref[...],
                         preferred_element_type=jnp.float32)


def _final_body(x_ref, a0_ref, a1_ref, w2_ref, b2_ref, o_ref):
    xv = x_ref[...]
    a = a0_ref[...] + a1_ref[...]
    h = jnp.dot(xv, w2_ref[:D, :], preferred_element_type=jnp.float32)
    h = h + jnp.dot(a, w2_ref[D:, :], preferred_element_type=jnp.float32)
    h = h + b2_ref[...]
    o_ref[...] = jnp.maximum(h, 0.0) + xv


def _make_sc_edge(cpt: int):
    mesh = plsc.VectorSubcoreMesh(core_axis_name="c", subcore_axis_name="s")

    @functools.partial(
        pl.kernel,
        mesh=mesh,
        out_type=jax.ShapeDtypeStruct((NC, AGG_PAD_ROWS, D), jnp.float32),
        scratch_types=[
            pltpu.VMEM((CHUNK,), jnp.int32),
            pltpu.VMEM((CHUNK,), jnp.int32),
            pltpu.VMEM((CHUNK, D), jnp.float32),
            pltpu.VMEM((CHUNK, D), jnp.float32),
            pltpu.VMEM((CHUNK, D), jnp.float32),
            pltpu.VMEM_SHARED((AGG_PAD_ROWS, D), jnp.float32),
            pltpu.SemaphoreType.DMA,
            pltpu.SemaphoreType.DMA,
            pltpu.SemaphoreType.DMA,
        ],
    )
    def sc_edge(src_hbm, dst_hbm, psrc_hbm, pdst_hbm, eproj_hbm, zeros_hbm,
                out_hbm, sidx, didx, abuf, bbuf, ebuf, agg_sh,
                sem0, sem1, sem2):
        c = lax.axis_index("c")
        s = lax.axis_index("s")
        wid = c * NS + s
        zr = AGG_PAD_ROWS // NS
        pltpu.sync_copy(zeros_hbm.at[pl.ds(s * zr, zr)],
                        agg_sh.at[pl.ds(s * zr, zr)])
        plsc.subcore_barrier()

        def chunk_body(i, carry):
            base = (wid * cpt + i) * CHUNK
            pltpu.sync_copy(src_hbm.at[pl.ds(base, CHUNK)], sidx)
            pltpu.sync_copy(dst_hbm.at[pl.ds(base, CHUNK)], didx)
            cp0 = pltpu.async_copy(psrc_hbm.at[sidx], abuf, sem0)
            cp1 = pltpu.async_copy(pdst_hbm.at[didx], bbuf, sem1)
            cp2 = pltpu.async_copy(eproj_hbm.at[pl.ds(base, CHUNK)], ebuf, sem2)
            cp0.wait()
            cp1.wait()
            cp2.wait()

            def row_body(r, carry2):
                for k in range(D // L):
                    sl = pl.ds(k * L, L)
                    v = abuf[r, sl] + bbuf[r, sl] + ebuf[r, sl]
                    abuf[r, sl] = jnp.maximum(v, 0.0)
                return carry2

            lax.fori_loop(0, CHUNK, row_body, 0, unroll=2)
            pltpu.sync_copy(abuf, agg_sh.at[didx], add=True)
            return carry

        lax.fori_loop(0, cpt, chunk_body, 0)
        plsc.subcore_barrier()
        pltpu.sync_copy(agg_sh.at[pl.ds(s * zr, zr)],
                        out_hbm.at[c, pl.ds(s * zr, zr)])

    return sc_edge


def kernel(x, edge_index, edge_attr, W1, b1, W2, b2):
    n_nodes = x.shape[0]
    n_edges = edge_index.shape[1]

    cpt = -(-n_edges // (NW * CHUNK))
    e_pad = NW * cpt * CHUNK
    pad = e_pad - n_edges
    src = edge_index[0].astype(jnp.int32)
    dst = edge_index[1].astype(jnp.int32)
    src_p = jnp.concatenate([src, jnp.zeros((pad,), jnp.int32)])
    dst_p = jnp.concatenate([dst, jnp.full((pad,), n_nodes, jnp.int32)])
    ea_p = jnp.concatenate(
        [edge_attr, jnp.zeros((pad, D_EDGE), edge_attr.dtype)])
    zeros = jnp.zeros((AGG_PAD_ROWS, D), jnp.float32)

    psrc, pdst = pl.pallas_call(
        _proj_body,
        out_shape=[jax.ShapeDtypeStruct((n_nodes, D), jnp.float32)] * 2,
    )(x, jnp.concatenate([W1[:D, :], W1[D:2 * D, :]], axis=1),
      b1.reshape(1, D))

    eblk = 4096
    eproj = pl.pallas_call(
        _eproj_body,
        grid=(e_pad // eblk,),
        in_specs=[
            pl.BlockSpec((eblk, D_EDGE), lambda i: (i, 0)),
            pl.BlockSpec((D_EDGE, D), lambda i: (0, 0)),
        ],
        out_specs=pl.BlockSpec((eblk, D), lambda i: (i, 0)),
        out_shape=jax.ShapeDtypeStruct((e_pad, D), jnp.float32),
    )(ea_p, W1[2 * D:, :])

    agg_parts = _make_sc_edge(cpt)(src_p, dst_p, psrc, pdst, eproj, zeros)

    nblk = 1000
    out = pl.pallas_call(
        _final_body,
        grid=(n_nodes // nblk,),
        in_specs=[
            pl.BlockSpec((nblk, D), lambda i: (i, 0)),
            pl.BlockSpec((nblk, D), lambda i: (i, 0)),
            pl.BlockSpec((nblk, D), lambda i: (i, 0)),
            pl.BlockSpec((2 * D, D), lambda i: (0, 0)),
            pl.BlockSpec((1, D), lambda i: (0, 0)),
        ],
        out_specs=pl.BlockSpec((nblk, D), lambda i: (i, 0)),
        out_shape=jax.ShapeDtypeStruct((n_nodes, D), jnp.float32),
    )(x, agg_parts[0, :n_nodes], agg_parts[1, :n_nodes],
      W2, b2.reshape(1, D))
    return out


# speedup vs baseline: 2.8231x; 1.3142x over previous
import functools

import jax
import jax.numpy as jnp
from jax import lax
from jax.experimental import pallas as pl
from jax.experimental.pallas import tpu as pltpu
from jax.experimental.pallas import tpu_sc as plsc

D = 128
D_EDGE = 4
NC = 2
NS = 16
L = 16
NW = NC * NS
CHUNK = 64
AGG_PAD_ROWS = 10112


def _proj_body(x_ref, w_ref, b1_ref, ps_ref, pd_ref):
    xv = x_ref[...]
    p = jnp.dot(xv, w_ref[...], preferred_element_type=jnp.float32)
    ps_ref[...] = p[:, :D]
    pd_ref[...] = p[:, D:] + b1_ref[...]


def _eproj_body(ea_ref, w_ref, o_ref):
    o_ref[...] = jnp.dot(ea_ref[...], w_ref[...],
                         preferred_element_type=jnp.float32)


def _final_body(x_ref, a0_ref, a1_ref, w2_ref, b2_ref, o_ref):
    xv = x_ref[...]
    a = a0_ref[...] + a1_ref[...]
    h = jnp.dot(xv, w2_ref[:D, :], preferred_element_type=jnp.float32)
    h = h + jnp.dot(a, w2_ref[D:, :], preferred_element_type=jnp.float32)
    h = h + b2_ref[...]
    o_ref[...] = jnp.maximum(h, 0.0) + xv


def _make_sc_edge(cpt: int):
    mesh = plsc.VectorSubcoreMesh(core_axis_name="c", subcore_axis_name="s")

    @functools.partial(
        pl.kernel,
        mesh=mesh,
        out_type=jax.ShapeDtypeStruct((NC, AGG_PAD_ROWS, D), jnp.float32),
        scratch_types=[
            pltpu.VMEM((2, CHUNK), jnp.int32),
            pltpu.VMEM((2, CHUNK), jnp.int32),
            pltpu.VMEM((2, CHUNK, D), jnp.float32),
            pltpu.VMEM((2, CHUNK, D), jnp.float32),
            pltpu.VMEM((2, CHUNK, D), jnp.float32),
            pltpu.VMEM_SHARED((AGG_PAD_ROWS, D), jnp.float32),
            pltpu.SemaphoreType.DMA,
            pltpu.SemaphoreType.DMA,
            pltpu.SemaphoreType.DMA,
            pltpu.SemaphoreType.DMA,
        ],
    )
    def sc_edge(src_hbm, dst_hbm, psrc_hbm, pdst_hbm, eproj_hbm, zeros_hbm,
                out_hbm, sidx, didx, abuf, bbuf, ebuf, agg_sh,
                gsem0, gsem1, isem0, isem1):
        c = lax.axis_index("c")
        s = lax.axis_index("s")
        wid = c * NS + s
        gsems = (gsem0, gsem1)
        isems = (isem0, isem1)
        zr = AGG_PAD_ROWS // NS
        pltpu.sync_copy(zeros_hbm.at[pl.ds(s * zr, zr)],
                        agg_sh.at[pl.ds(s * zr, zr)])

        def idx_issue(ch, b):
            base = (wid * cpt + ch) * CHUNK
            pltpu.async_copy(src_hbm.at[pl.ds(base, CHUNK)], sidx.at[b],
                             isems[b])
            pltpu.async_copy(dst_hbm.at[pl.ds(base, CHUNK)], didx.at[b],
                             isems[b])

        def idx_wait(b):
            pltpu.make_async_copy(src_hbm.at[pl.ds(0, CHUNK)], sidx.at[b],
                                  isems[b]).wait()
            pltpu.make_async_copy(dst_hbm.at[pl.ds(0, CHUNK)], didx.at[b],
                                  isems[b]).wait()

        def gather_issue(ch, b):
            base = (wid * cpt + ch) * CHUNK
            pltpu.async_copy(psrc_hbm.at[sidx.at[b]], abuf.at[b], gsems[b])
            pltpu.async_copy(pdst_hbm.at[didx.at[b]], bbuf.at[b], gsems[b])
            pltpu.async_copy(eproj_hbm.at[pl.ds(base, CHUNK)], ebuf.at[b],
                             gsems[b])

        def gather_wait(b):
            pltpu.make_async_copy(psrc_hbm.at[sidx.at[b]], abuf.at[b],
                                  gsems[b]).wait()
            pltpu.make_async_copy(pdst_hbm.at[didx.at[b]], bbuf.at[b],
                                  gsems[b]).wait()
            pltpu.make_async_copy(eproj_hbm.at[pl.ds(0, CHUNK)], ebuf.at[b],
                                  gsems[b]).wait()

        base0 = wid * cpt * CHUNK
        pltpu.sync_copy(src_hbm.at[pl.ds(base0, CHUNK)], sidx.at[0])
        pltpu.sync_copy(dst_hbm.at[pl.ds(base0, CHUNK)], didx.at[0])
        idx_issue(1, 1)
        gather_issue(0, 0)

        def body(g, carry):
            for b in range(2):
                ch = 2 * g + b
                gather_wait(b)

                @pl.when(ch + 1 < cpt)
                def _():
                    idx_wait(1 - b)
                    gather_issue(ch + 1, 1 - b)

                def row_body(r, carry2):
                    for k in range(D // L):
                        sl = pl.ds(k * L, L)
                        v = abuf[b, r, sl] + bbuf[b, r, sl] + ebuf[b, r, sl]
                        abuf[b, r, sl] = jnp.maximum(v, 0.0)
                    return carry2

                lax.fori_loop(0, CHUNK, row_body, 0, unroll=2)
                pltpu.sync_copy(abuf.at[b], agg_sh.at[didx.at[b]], add=True)

                @pl.when(ch + 2 < cpt)
                def _():
                    idx_issue(ch + 2, b)

            return carry

        lax.fori_loop(0, cpt // 2, body, 0)
        plsc.subcore_barrier()
        pltpu.sync_copy(agg_sh.at[pl.ds(s * zr, zr)],
                        out_hbm.at[c, pl.ds(s * zr, zr)])

    return sc_edge


def kernel(x, edge_index, edge_attr, W1, b1, W2, b2):
    n_nodes = x.shape[0]
    n_edges = edge_index.shape[1]

    cpt = -(-n_edges // (NW * CHUNK))
    cpt = cpt + (cpt % 2)
    e_pad = NW * cpt * CHUNK
    pad = e_pad - n_edges
    src = edge_index[0].astype(jnp.int32)
    dst = edge_index[1].astype(jnp.int32)
    src_p = jnp.concatenate([src, jnp.zeros((pad,), jnp.int32)])
    dst_p = jnp.concatenate([dst, jnp.full((pad,), n_nodes, jnp.int32)])
    ea_p = jnp.concatenate(
        [edge_attr, jnp.zeros((pad, D_EDGE), edge_attr.dtype)])
    zeros = jnp.zeros((AGG_PAD_ROWS, D), jnp.float32)

    psrc, pdst = pl.pallas_call(
        _proj_body,
        out_shape=[jax.ShapeDtypeStruct((n_nodes, D), jnp.float32)] * 2,
    )(x, jnp.concatenate([W1[:D, :], W1[D:2 * D, :]], axis=1),
      b1.reshape(1, D))

    eblk = 4096
    eproj = pl.pallas_call(
        _eproj_body,
        grid=(e_pad // eblk,),
        in_specs=[
            pl.BlockSpec((eblk, D_EDGE), lambda i: (i, 0)),
            pl.BlockSpec((D_EDGE, D), lambda i: (0, 0)),
        ],
        out_specs=pl.BlockSpec((eblk, D), lambda i: (i, 0)),
        out_shape=jax.ShapeDtypeStruct((e_pad, D), jnp.float32),
    )(ea_p, W1[2 * D:, :])

    agg_parts = _make_sc_edge(cpt)(src_p, dst_p, psrc, pdst, eproj, zeros)

    nblk = 1000
    out = pl.pallas_call(
        _final_body,
        grid=(n_nodes // nblk,),
        in_specs=[
            pl.BlockSpec((nblk, D), lambda i: (i, 0)),
            pl.BlockSpec((nblk, D), lambda i: (i, 0)),
            pl.BlockSpec((nblk, D), lambda i: (i, 0)),
            pl.BlockSpec((2 * D, D), lambda i: (0, 0)),
            pl.BlockSpec((1, D), lambda i: (0, 0)),
        ],
        out_specs=pl.BlockSpec((nblk, D), lambda i: (i, 0)),
        out_shape=jax.ShapeDtypeStruct((n_nodes, D), jnp.float32),
    )(x, agg_parts[0, :n_nodes], agg_parts[1, :n_nodes],
      W2, b2.reshape(1, D))
    return out


# speedup vs baseline: 2.9217x; 1.0349x over previous
import functools

import jax
import jax.numpy as jnp
from jax import lax
from jax.experimental import pallas as pl
from jax.experimental.pallas import tpu as pltpu
from jax.experimental.pallas import tpu_sc as plsc

D = 128
D_EDGE = 4
NC = 2
NS = 16
L = 16
NW = NC * NS
CHUNK = 64
AGG_PAD_ROWS = 10112


def _proj_body(x_ref, w_ref, b1_ref, ps_ref, pd_ref):
    xv = x_ref[...]
    p = jnp.dot(xv, w_ref[...], preferred_element_type=jnp.float32)
    ps_ref[...] = p[:, :D]
    pd_ref[...] = p[:, D:] + b1_ref[...]


def _eproj_body(ea_ref, w_ref, o_ref):
    o_ref[...] = jnp.dot(ea_ref[...], w_ref[...],
                         preferred_element_type=jnp.float32)


def _final_body(x_ref, a0_ref, a1_ref, w2_ref, b2_ref, o_ref):
    xv = x_ref[...]
    a = a0_ref[...] + a1_ref[...]
    h = jnp.dot(xv, w2_ref[:D, :], preferred_element_type=jnp.float32)
    h = h + jnp.dot(a, w2_ref[D:, :], preferred_element_type=jnp.float32)
    h = h + b2_ref[...]
    o_ref[...] = jnp.maximum(h, 0.0) + xv


def _make_sc_edge(cpt: int):
    mesh = plsc.VectorSubcoreMesh(core_axis_name="c", subcore_axis_name="s")

    @functools.partial(
        pl.kernel,
        mesh=mesh,
        out_type=jax.ShapeDtypeStruct((NC, AGG_PAD_ROWS, D), jnp.float32),
        scratch_types=[
            pltpu.VMEM((2, CHUNK), jnp.int32),
            pltpu.VMEM((2, CHUNK), jnp.int32),
            pltpu.VMEM((2, CHUNK), jnp.int32),
            pltpu.VMEM((2, CHUNK, D), jnp.float32),
            pltpu.VMEM((2, CHUNK, D), jnp.float32),
            pltpu.VMEM((2, CHUNK, D), jnp.float32),
            pltpu.VMEM_SHARED((AGG_PAD_ROWS, D), jnp.float32),
            pltpu.SemaphoreType.DMA,
            pltpu.SemaphoreType.DMA,
            pltpu.SemaphoreType.DMA,
            pltpu.SemaphoreType.DMA,
            pltpu.SemaphoreType.DMA,
            pltpu.SemaphoreType.DMA,
        ],
    )
    def sc_edge(src_hbm, dst_hbm, psrc_hbm, pdst_hbm, eproj_hbm, zeros_hbm,
                out_hbm, sidx, didx, didx_s, abuf, bbuf, ebuf, agg_sh,
                gsem0, gsem1, isem0, isem1, ssem0, ssem1):
        c = lax.axis_index("c")
        s = lax.axis_index("s")
        wid = c * NS + s
        gsems = (gsem0, gsem1)
        isems = (isem0, isem1)
        ssems = (ssem0, ssem1)
        zr = AGG_PAD_ROWS // NS
        pltpu.sync_copy(zeros_hbm.at[pl.ds(s * zr, zr)],
                        agg_sh.at[pl.ds(s * zr, zr)])

        def idx_issue(ch, b):
            base = (wid * cpt + ch) * CHUNK
            pltpu.async_copy(src_hbm.at[pl.ds(base, CHUNK)], sidx.at[b],
                             isems[b])
            pltpu.async_copy(dst_hbm.at[pl.ds(base, CHUNK)], didx.at[b],
                             isems[b])

        def idx_wait(b):
            pltpu.make_async_copy(src_hbm.at[pl.ds(0, CHUNK)], sidx.at[b],
                                  isems[b]).wait()
            pltpu.make_async_copy(dst_hbm.at[pl.ds(0, CHUNK)], didx.at[b],
                                  isems[b]).wait()

        def gather_issue(ch, b):
            base = (wid * cpt + ch) * CHUNK
            pltpu.async_copy(psrc_hbm.at[sidx.at[b]], abuf.at[b], gsems[b])
            pltpu.async_copy(pdst_hbm.at[didx.at[b]], bbuf.at[b], gsems[b])
            pltpu.async_copy(eproj_hbm.at[pl.ds(base, CHUNK)], ebuf.at[b],
                             gsems[b])

        def gather_wait(b):
            pltpu.make_async_copy(psrc_hbm.at[sidx.at[b]], abuf.at[b],
                                  gsems[b]).wait()
            pltpu.make_async_copy(pdst_hbm.at[didx.at[b]], bbuf.at[b],
                                  gsems[b]).wait()
            pltpu.make_async_copy(eproj_hbm.at[pl.ds(0, CHUNK)], ebuf.at[b],
                                  gsems[b]).wait()

        base0 = wid * cpt * CHUNK
        pltpu.sync_copy(src_hbm.at[pl.ds(base0, CHUNK)], sidx.at[0])
        pltpu.sync_copy(dst_hbm.at[pl.ds(base0, CHUNK)], didx.at[0])
        idx_issue(1, 1)
        gather_issue(0, 0)

        def scatter_wait(b):
            pltpu.make_async_copy(abuf.at[b], agg_sh.at[didx_s.at[b]],
                                  ssems[b]).wait()

        def body(g, carry):
            for b in range(2):
                ch = 2 * g + b
                gather_wait(b)

                @pl.when(jnp.logical_and(ch >= 1, ch + 1 < cpt))
                def _():
                    scatter_wait(1 - b)

                @pl.when(ch + 1 < cpt)
                def _():
                    idx_wait(1 - b)
                    gather_issue(ch + 1, 1 - b)

                def row_body(r, carry2):
                    for k in range(D // L):
                        sl = pl.ds(k * L, L)
                        v = abuf[b, r, sl] + bbuf[b, r, sl] + ebuf[b, r, sl]
                        abuf[b, r, sl] = jnp.maximum(v, 0.0)
                    return carry2

                lax.fori_loop(0, CHUNK, row_body, 0, unroll=2)
                for k in range(CHUNK // L):
                    sl = pl.ds(k * L, L)
                    didx_s[b, sl] = didx[b, sl]
                pltpu.async_copy(abuf.at[b], agg_sh.at[didx_s.at[b]],
                                 ssems[b], add=True)

                @pl.when(ch + 2 < cpt)
                def _():
                    idx_issue(ch + 2, b)

            return carry

        lax.fori_loop(0, cpt // 2, body, 0)
        scatter_wait(0)
        scatter_wait(1)
        plsc.subcore_barrier()
        pltpu.sync_copy(agg_sh.at[pl.ds(s * zr, zr)],
                        out_hbm.at[c, pl.ds(s * zr, zr)])

    return sc_edge


def kernel(x, edge_index, edge_attr, W1, b1, W2, b2):
    n_nodes = x.shape[0]
    n_edges = edge_index.shape[1]

    cpt = -(-n_edges // (NW * CHUNK))
    cpt = cpt + (cpt % 2)
    e_pad = NW * cpt * CHUNK
    pad = e_pad - n_edges
    src = edge_index[0].astype(jnp.int32)
    dst = edge_index[1].astype(jnp.int32)
    src_p = jnp.concatenate([src, jnp.zeros((pad,), jnp.int32)])
    dst_p = jnp.concatenate([dst, jnp.full((pad,), n_nodes, jnp.int32)])
    ea_p = jnp.concatenate(
        [edge_attr, jnp.zeros((pad, D_EDGE), edge_attr.dtype)])
    zeros = jnp.zeros((AGG_PAD_ROWS, D), jnp.float32)

    psrc, pdst = pl.pallas_call(
        _proj_body,
        out_shape=[jax.ShapeDtypeStruct((n_nodes, D), jnp.float32)] * 2,
    )(x, jnp.concatenate([W1[:D, :], W1[D:2 * D, :]], axis=1),
      b1.reshape(1, D))

    eblk = 4096
    eproj = pl.pallas_call(
        _eproj_body,
        grid=(e_pad // eblk,),
        in_specs=[
            pl.BlockSpec((eblk, D_EDGE), lambda i: (i, 0)),
            pl.BlockSpec((D_EDGE, D), lambda i: (0, 0)),
        ],
        out_specs=pl.BlockSpec((eblk, D), lambda i: (i, 0)),
        out_shape=jax.ShapeDtypeStruct((e_pad, D), jnp.float32),
    )(ea_p, W1[2 * D:, :])

    agg_parts = _make_sc_edge(cpt)(src_p, dst_p, psrc, pdst, eproj, zeros)

    nblk = 1000
    out = pl.pallas_call(
        _final_body,
        grid=(n_nodes // nblk,),
        in_specs=[
            pl.BlockSpec((nblk, D), lambda i: (i, 0)),
            pl.BlockSpec((nblk, D), lambda i: (i, 0)),
            pl.BlockSpec((nblk, D), lambda i: (i, 0)),
            pl.BlockSpec((2 * D, D), lambda i: (0, 0)),
            pl.BlockSpec((1, D), lambda i: (0, 0)),
        ],
        out_specs=pl.BlockSpec((nblk, D), lambda i: (i, 0)),
        out_shape=jax.ShapeDtypeStruct((n_nodes, D), jnp.float32),
    )(x, agg_parts[0, :n_nodes], agg_parts[1, :n_nodes],
      W2, b2.reshape(1, D))
    return out


# speedup vs baseline: 3.2005x; 1.0954x over previous
import functools

import jax
import jax.numpy as jnp
from jax import lax
from jax.experimental import pallas as pl
from jax.experimental.pallas import tpu as pltpu
from jax.experimental.pallas import tpu_sc as plsc

D = 128
D_EDGE = 4
NC = 2
NS = 16
L = 16
NW = NC * NS
CHUNK = 64
AGG_PAD_ROWS = 10112


def _pack_bf16_halves(p):
    pr = p.astype(jnp.bfloat16).astype(jnp.float32)
    u = jax.lax.bitcast_convert_type(pr, jnp.uint32)
    lo = u[:, : D // 2] >> 16
    hi = u[:, D // 2:] & jnp.uint32(0xFFFF0000)
    return jax.lax.bitcast_convert_type(lo | hi, jnp.int32)


def _proj_body(x_ref, w_ref, b1_ref, ps_ref, pd_ref):
    xv = x_ref[...]
    p = jnp.dot(xv, w_ref[...], preferred_element_type=jnp.float32)
    ps_ref[...] = _pack_bf16_halves(p[:, :D])
    pd_ref[...] = _pack_bf16_halves(p[:, D:] + b1_ref[...])


def _eproj_body(ea_ref, w_ref, o_ref):
    o_ref[...] = _pack_bf16_halves(
        jnp.dot(ea_ref[...], w_ref[...], preferred_element_type=jnp.float32))


def _final_body(x_ref, a0_ref, a1_ref, w2_ref, b2_ref, o_ref):
    xv = x_ref[...]
    a = a0_ref[...] + a1_ref[...]
    h = jnp.dot(xv, w2_ref[:D, :], preferred_element_type=jnp.float32)
    h = h + jnp.dot(a, w2_ref[D:, :], preferred_element_type=jnp.float32)
    h = h + b2_ref[...]
    o_ref[...] = jnp.maximum(h, 0.0) + xv


def _make_sc_edge(cpt: int):
    mesh = plsc.VectorSubcoreMesh(core_axis_name="c", subcore_axis_name="s")

    @functools.partial(
        pl.kernel,
        mesh=mesh,
        compiler_params=pltpu.CompilerParams(needs_layout_passes=False,
                                             use_tc_tiling_on_sc=False),
        out_type=jax.ShapeDtypeStruct((NC, AGG_PAD_ROWS, D), jnp.float32),
        scratch_types=[
            pltpu.VMEM((2, CHUNK), jnp.int32),
            pltpu.VMEM((2, CHUNK), jnp.int32),
            pltpu.VMEM((2, CHUNK), jnp.int32),
            pltpu.VMEM((2, CHUNK, D // 2), jnp.int32),
            pltpu.VMEM((2, CHUNK, D // 2), jnp.int32),
            pltpu.VMEM((2, CHUNK, D // 2), jnp.int32),
            pltpu.VMEM((2, CHUNK, D), jnp.float32),
            pltpu.VMEM_SHARED((AGG_PAD_ROWS, D), jnp.float32),
            pltpu.SemaphoreType.DMA,
            pltpu.SemaphoreType.DMA,
            pltpu.SemaphoreType.DMA,
            pltpu.SemaphoreType.DMA,
            pltpu.SemaphoreType.DMA,
            pltpu.SemaphoreType.DMA,
        ],
    )
    def sc_edge(src_hbm, dst_hbm, psrc_hbm, pdst_hbm, eproj_hbm, zeros_hbm,
                out_hbm, sidx, didx, didx_s, abuf, bbuf, ebuf, mbuf, agg_sh,
                gsem0, gsem1, isem0, isem1, ssem0, ssem1):
        c = lax.axis_index("c")
        s = lax.axis_index("s")
        wid = c * NS + s
        gsems = (gsem0, gsem1)
        isems = (isem0, isem1)
        ssems = (ssem0, ssem1)
        zr = AGG_PAD_ROWS // NS
        pltpu.sync_copy(zeros_hbm.at[pl.ds(s * zr, zr)],
                        agg_sh.at[pl.ds(s * zr, zr)])

        def idx_issue(ch, b):
            base = (wid * cpt + ch) * CHUNK
            pltpu.async_copy(src_hbm.at[pl.ds(base, CHUNK)], sidx.at[b],
                             isems[b])
            pltpu.async_copy(dst_hbm.at[pl.ds(base, CHUNK)], didx.at[b],
                             isems[b])

        def idx_wait(b):
            pltpu.make_async_copy(src_hbm.at[pl.ds(0, CHUNK)], sidx.at[b],
                                  isems[b]).wait()
            pltpu.make_async_copy(dst_hbm.at[pl.ds(0, CHUNK)], didx.at[b],
                                  isems[b]).wait()

        def gather_issue(ch, b):
            base = (wid * cpt + ch) * CHUNK
            pltpu.async_copy(psrc_hbm.at[sidx.at[b]], abuf.at[b], gsems[b])
            pltpu.async_copy(pdst_hbm.at[didx.at[b]], bbuf.at[b], gsems[b])
            pltpu.async_copy(eproj_hbm.at[pl.ds(base, CHUNK)], ebuf.at[b],
                             gsems[b])

        def gather_wait(b):
            pltpu.make_async_copy(psrc_hbm.at[sidx.at[b]], abuf.at[b],
                                  gsems[b]).wait()
            pltpu.make_async_copy(pdst_hbm.at[didx.at[b]], bbuf.at[b],
                                  gsems[b]).wait()
            pltpu.make_async_copy(eproj_hbm.at[pl.ds(0, CHUNK)], ebuf.at[b],
                                  gsems[b]).wait()

        base0 = wid * cpt * CHUNK
        pltpu.sync_copy(src_hbm.at[pl.ds(base0, CHUNK)], sidx.at[0])
        pltpu.sync_copy(dst_hbm.at[pl.ds(base0, CHUNK)], didx.at[0])
        idx_issue(1, 1)
        gather_issue(0, 0)

        def scatter_wait(b):
            pltpu.make_async_copy(mbuf.at[b], agg_sh.at[didx_s.at[b]],
                                  ssems[b]).wait()

        def body(g, carry):
            for b in range(2):
                ch = 2 * g + b
                gather_wait(b)

                @pl.when(ch + 1 < cpt)
                def _():
                    idx_wait(1 - b)
                    gather_issue(ch + 1, 1 - b)

                @pl.when(ch >= 2)
                def _():
                    scatter_wait(b)

                zero = jnp.zeros((2 * L,), jnp.bfloat16)

                def row_body(r, carry2):
                    for k in range(D // (2 * L)):
                        sl = pl.ds(k * L, L)
                        a = plsc.bitcast(abuf[b, r, sl], jnp.bfloat16)
                        bb = plsc.bitcast(bbuf[b, r, sl], jnp.bfloat16)
                        e = plsc.bitcast(ebuf[b, r, sl], jnp.bfloat16)
                        v = jnp.maximum(a + bb + e, zero)
                        vi = plsc.bitcast(v, jnp.int32)
                        lo = plsc.bitcast(vi << 16, jnp.float32)
                        hi = plsc.bitcast(vi & jnp.int32(-65536), jnp.float32)
                        mbuf[b, r, pl.ds(k * L, L)] = lo
                        mbuf[b, r, pl.ds(D // 2 + k * L, L)] = hi
                    return carry2

                lax.fori_loop(0, CHUNK, row_body, 0, unroll=2)
                for k in range(CHUNK // L):
                    sl = pl.ds(k * L, L)
                    didx_s[b, sl] = didx[b, sl]
                pltpu.async_copy(mbuf.at[b], agg_sh.at[didx_s.at[b]],
                                 ssems[b], add=True)

                @pl.when(ch + 2 < cpt)
                def _():
                    idx_issue(ch + 2, b)

            return carry

        lax.fori_loop(0, cpt // 2, body, 0)
        scatter_wait(0)
        scatter_wait(1)
        plsc.subcore_barrier()
        pltpu.sync_copy(agg_sh.at[pl.ds(s * zr, zr)],
                        out_hbm.at[c, pl.ds(s * zr, zr)])

    return sc_edge


def kernel(x, edge_index, edge_attr, W1, b1, W2, b2):
    n_nodes = x.shape[0]
    n_edges = edge_index.shape[1]

    cpt = -(-n_edges // (NW * CHUNK))
    cpt = cpt + (cpt % 2)
    e_pad = NW * cpt * CHUNK
    pad = e_pad - n_edges
    src = edge_index[0].astype(jnp.int32)
    dst = edge_index[1].astype(jnp.int32)
    src_p = jnp.concatenate([src, jnp.zeros((pad,), jnp.int32)])
    dst_p = jnp.concatenate([dst, jnp.full((pad,), n_nodes, jnp.int32)])
    ea_p = jnp.concatenate(
        [edge_attr, jnp.zeros((pad, D_EDGE), edge_attr.dtype)])
    zeros = jnp.zeros((AGG_PAD_ROWS, D), jnp.float32)

    psrc, pdst = pl.pallas_call(
        _proj_body,
        out_shape=[jax.ShapeDtypeStruct((n_nodes, D // 2), jnp.int32)] * 2,
    )(x, jnp.concatenate([W1[:D, :], W1[D:2 * D, :]], axis=1),
      b1.reshape(1, D))

    eblk = 4096
    eproj = pl.pallas_call(
        _eproj_body,
        grid=(e_pad // eblk,),
        in_specs=[
            pl.BlockSpec((eblk, D_EDGE), lambda i: (i, 0)),
            pl.BlockSpec((D_EDGE, D), lambda i: (0, 0)),
        ],
        out_specs=pl.BlockSpec((eblk, D // 2), lambda i: (i, 0)),
        out_shape=jax.ShapeDtypeStruct((e_pad, D // 2), jnp.int32),
    )(ea_p, W1[2 * D:, :])

    agg_parts = _make_sc_edge(cpt)(src_p, dst_p, psrc, pdst, eproj, zeros)


    nblk = 1000
    out = pl.pallas_call(
        _final_body,
        grid=(n_nodes // nblk,),
        in_specs=[
            pl.BlockSpec((nblk, D), lambda i: (i, 0)),
            pl.BlockSpec((nblk, D), lambda i: (i, 0)),
            pl.BlockSpec((nblk, D), lambda i: (i, 0)),
            pl.BlockSpec((2 * D, D), lambda i: (0, 0)),
            pl.BlockSpec((1, D), lambda i: (0, 0)),
        ],
        out_specs=pl.BlockSpec((nblk, D), lambda i: (i, 0)),
        out_shape=jax.ShapeDtypeStruct((n_nodes, D), jnp.float32),
    )(x, agg_parts[0, :n_nodes], agg_parts[1, :n_nodes],
      W2, b2.reshape(1, D))
    return out
